# baseline (speedup 1.0000x reference)
import functools

import jax
import jax.numpy as jnp
from jax import lax
from jax.experimental import pallas as pl
from jax.experimental.pallas import tpu as pltpu
from jax.experimental.pallas import tpu_sc as plsc

NC, NS = 2, 16
NW = NC * NS
BATCH, HIST, D = 4096, 50, 256
ROWS = 74
CPW = BATCH // NW
NPAIR = CPW // 2
GR = 48
NG = D // 16
GROUPS = ((0, range(16)), (16, range(16)), (32, range(16)), (34, (14, 15)))


@functools.partial(
    pl.kernel,
    out_type=jax.ShapeDtypeStruct((BATCH, HIST, D), jnp.float32),
    mesh=plsc.VectorSubcoreMesh(core_axis_name="c", subcore_axis_name="s"),
    scratch_types=[
        pltpu.VMEM((ROWS, D), jnp.float32),
        pltpu.VMEM((NPAIR, GR), jnp.int32),
        pltpu.VMEM((2 * NPAIR + 16,), jnp.int32),
        pltpu.VMEM((NPAIR, HIST), jnp.int32),
        pltpu.VMEM((2, HIST, D), jnp.float32),
        pltpu.VMEM((2, HIST, D), jnp.float32),
    ]
    + [pltpu.SemaphoreType.DMA] * 2
    + [pltpu.SemaphoreType.DMA] * 4,
)
def _gather_kernel(
    table_hbm, gidx_hbm, tidx_hbm, bidx_hbm, out_hbm,
    table_v, gidx_v, tidx_v, bidx_v, gslot, bslot, *sems
):
    gsem, ssem = sems[:2], sems[2:]
    wid = lax.axis_index("s") * NC + lax.axis_index("c")
    base = wid * CPW
    pltpu.sync_copy(table_hbm, table_v)
    pltpu.sync_copy(gidx_hbm.at[wid], gidx_v)
    pltpu.sync_copy(tidx_hbm.at[wid], tidx_v)
    pltpu.sync_copy(bidx_hbm.at[wid], bidx_v)

    def start_gather(p, s):
        pltpu.async_copy(
            table_hbm.at[gidx_v.at[p]], gslot.at[s, pl.ds(0, GR)], gsem[s]
        )

    def wait_gather(s):
        pltpu.make_async_copy(
            table_hbm.at[gidx_v.at[0]], gslot.at[s, pl.ds(0, GR)], gsem[s]
        ).wait()

    def copy_row(a, slot_ref, s, i):
        vals = [table_v[a, pl.ds(16 * j, 16)] for j in range(NG)]
        for j in range(NG):
            slot_ref[s, i, pl.ds(16 * j, 16)] = vals[j]

    def build_tails(p, s):
        ivec = tidx_v[pl.ds(2 * p, 16)]
        copy_row(ivec[0], gslot, s, GR)
        copy_row(ivec[1], gslot, s, GR + 1)

    def build_entry(p, s):
        for off, lanes in GROUPS:
            ivec = bidx_v[p, pl.ds(off, 16)]
            for l in lanes:
                copy_row(ivec[l], bslot, s, off + l)

    def start_scatter(slot_ref, s, sem_i, c):
        pltpu.async_copy(slot_ref.at[s], out_hbm.at[base + c], ssem[sem_i])

    def wait_scatter(slot_ref, s, sem_i):
        pltpu.make_async_copy(
            slot_ref.at[s], out_hbm.at[base], ssem[sem_i]
        ).wait()

    def pair(p, s, first):
        cg, cb = 2 * p, 2 * p + 1
        if not first:
            wait_scatter(gslot, s, s)
        start_gather(p, s)
        if not first:
            wait_scatter(bslot, s, 2 + s)
        build_entry(p, s)
        start_scatter(bslot, s, 2 + s, cb)
        build_tails(p, s)
        wait_gather(s)
        start_scatter(gslot, s, s, cg)

    pair(0, 0, True)
    pair(1, 1, True)

    def body(k, carry):
        pair(2 * k, 0, False)
        pair(2 * k + 1, 1, False)
        return carry

    lax.fori_loop(1, NPAIR // 2, body, 0)

    for s in range(2):
        wait_scatter(gslot, s, s)
        wait_scatter(bslot, s, 2 + s)


def kernel(actions, action_embeddings):
    a3 = actions.reshape(NW, CPW, HIST).astype(jnp.int32)
    gidx = a3[:, ::2, :GR]
    tidx = a3[:, ::2, GR:].reshape(NW, 2 * NPAIR)
    tidx = jnp.pad(tidx, ((0, 0), (0, 16)))
    bidx = a3[:, 1::2, :]
    return _gather_kernel(action_embeddings, gidx, tidx, bidx)


# speedup vs baseline: 1.4150x; 1.4150x over previous
import functools

import jax
import jax.numpy as jnp
from jax import lax
from jax.experimental import pallas as pl
from jax.experimental.pallas import tpu as pltpu
from jax.experimental.pallas import tpu_sc as plsc

NC, NS = 2, 16
NW = NC * NS
BATCH, HIST, D = 4096, 50, 256
ROWS = 74
CPW = BATCH // NW
NBUF = 4
NG = D // 16


@functools.partial(
    pl.kernel,
    out_type=jax.ShapeDtypeStruct((BATCH, HIST, D), jnp.float32),
    mesh=plsc.VectorSubcoreMesh(core_axis_name="c", subcore_axis_name="s"),
    scratch_types=[
        pltpu.VMEM((ROWS, D), jnp.float32),
        pltpu.VMEM((CPW, HIST), jnp.int32),
        pltpu.VMEM((NBUF, HIST, D), jnp.float32),
    ]
    + [pltpu.SemaphoreType.DMA] * NBUF,
)
def _gather_kernel(table_hbm, idx_hbm, out_hbm, table_v, idx_v, rows_v, *ssem):
    wid = lax.axis_index("s") * NC + lax.axis_index("c")
    base = wid * CPW
    pltpu.sync_copy(table_hbm, table_v)
    pltpu.sync_copy(idx_hbm.at[wid], idx_v)

    def copy_row(a, b, i):
        vals = [table_v[a, pl.ds(16 * j, 16)] for j in range(NG)]
        for j in range(NG):
            rows_v[b, i, pl.ds(16 * j, 16)] = vals[j]

    def build(c, b):
        def group(g, carry):
            ivec = idx_v[c, pl.ds(16 * g, 16)]
            for l in range(16):
                copy_row(ivec[l], b, 16 * g + l)
            return carry

        lax.fori_loop(0, 3, group, 0)
        tvec = idx_v[c, pl.ds(34, 16)]
        copy_row(tvec[14], b, 48)
        copy_row(tvec[15], b, 49)

    def start_scatter(c, b):
        pltpu.async_copy(rows_v.at[b], out_hbm.at[base + c], ssem[b])

    def wait_scatter(b):
        pltpu.make_async_copy(rows_v.at[b], out_hbm.at[base], ssem[b]).wait()

    for b in range(NBUF):
        build(b, b)
        start_scatter(b, b)

    def body(k, carry):
        for b in range(NBUF):
            c = k * NBUF + b
            wait_scatter(b)
            build(c, b)
            start_scatter(c, b)
        return carry

    lax.fori_loop(1, CPW // NBUF, body, 0)

    for b in range(NBUF):
        wait_scatter(b)


def kernel(actions, action_embeddings):
    idx = actions.reshape(NW, CPW, HIST).astype(jnp.int32)
    return _gather_kernel(action_embeddings, idx)
